# Initial kernel scaffold; baseline (speedup 1.0000x reference)
import jax
import jax.numpy as jnp
from jax.experimental import pallas as pl


def kernel(features, edge_index):
    raise NotImplementedError("write your pallas kernel here")


# speedup vs baseline: 7.5066x; 7.5066x over previous
import functools

import jax
import jax.numpy as jnp
from jax import lax
from jax.experimental import pallas as pl
from jax.experimental.pallas import tpu as pltpu
from jax.experimental.pallas import tpu_sc as plsc

N = 10000
E = 320000
D = 128

NC = 2
NS = 16
NW = NC * NS

NPAD = 10240
RT = NPAD // NS
K = 80
EW = E // NW
CW = EW // K

RB = 1024
NBLK = NPAD // RB


def _agg_body(x_hbm, src_hbm, dst_hbm, zrows_hbm, zdeg_hbm, ones_hbm,
              p_out, degp_out,
              acc_sh, deg_sh, srcbuf, dstbuf, rows_v, ones_v):
    cid = lax.axis_index("c")
    sid = lax.axis_index("s")
    wid = sid * NC + cid

    pltpu.sync_copy(zrows_hbm, acc_sh.at[pl.ds(sid * RT, RT)])
    pltpu.sync_copy(zdeg_hbm, deg_sh.at[pl.ds(sid * RT, RT)])
    pltpu.sync_copy(ones_hbm, ones_v)
    pltpu.sync_copy(src_hbm.at[wid], srcbuf)
    pltpu.sync_copy(dst_hbm.at[wid], dstbuf)
    plsc.subcore_barrier()

    def chunk(ci, carry):
        pltpu.sync_copy(x_hbm.at[srcbuf.at[ci]], rows_v)
        pltpu.sync_copy(rows_v, acc_sh.at[dstbuf.at[ci]], add=True)
        pltpu.sync_copy(ones_v, deg_sh.at[dstbuf.at[ci]], add=True)
        return carry

    lax.fori_loop(0, CW, chunk, 0)
    plsc.subcore_barrier()

    pltpu.sync_copy(acc_sh.at[pl.ds(sid * RT, RT)],
                    p_out.at[cid, pl.ds(sid * RT, RT)])
    pltpu.sync_copy(deg_sh.at[pl.ds(sid * RT, RT)],
                    degp_out.at[cid, pl.ds(sid * RT, RT)])


_sc_agg = functools.partial(
    pl.kernel,
    out_type=(jax.ShapeDtypeStruct((NC, NPAD, D), jnp.float32),
              jax.ShapeDtypeStruct((NC, NPAD), jnp.float32)),
    mesh=plsc.VectorSubcoreMesh(core_axis_name="c", subcore_axis_name="s"),
    scratch_types=(
        pltpu.VMEM_SHARED((NPAD, D), jnp.float32),
        pltpu.VMEM_SHARED((NPAD,), jnp.float32),
        pltpu.VMEM((CW, K), jnp.int32),
        pltpu.VMEM((CW, K), jnp.int32),
        pltpu.VMEM((K, D), jnp.float32),
        pltpu.VMEM((K,), jnp.float32),
    ),
)(_agg_body)


def _combine_body(p_ref, degp_ref, x1_ref):
    i = pl.program_id(0)
    deg = degp_ref[0, pl.ds(i * RB, RB)] + degp_ref[1, pl.ds(i * RB, RB)]
    rec = 1.0 / jnp.maximum(deg, 1.0)
    x1_ref[...] = (p_ref[0] + p_ref[1]) * rec[:, None]


def _tc_combine(p, degp):
    return pl.pallas_call(
        _combine_body,
        grid=(NBLK,),
        in_specs=[
            pl.BlockSpec((NC, RB, D), lambda i: (0, i, 0)),
            pl.BlockSpec((NC, NPAD), lambda i: (0, 0)),
        ],
        out_specs=pl.BlockSpec((RB, D), lambda i: (i, 0)),
        out_shape=jax.ShapeDtypeStruct((NPAD, D), jnp.float32),
    )(p, degp)


def _final_body(p_ref, degp_ref, x1_ref, out_ref):
    i = pl.program_id(0)
    deg = degp_ref[0, pl.ds(i * RB, RB)] + degp_ref[1, pl.ds(i * RB, RB)]
    rec = 1.0 / jnp.maximum(deg, 1.0)
    x2 = (p_ref[0] + p_ref[1]) * rec[:, None]
    x1 = x1_ref[...]
    dot = jnp.sum(x1 * x2, axis=1, keepdims=True)
    n1 = jnp.sqrt(jnp.sum(x1 * x1, axis=1, keepdims=True))
    n2 = jnp.sqrt(jnp.sum(x2 * x2, axis=1, keepdims=True))
    w = dot / (jnp.maximum(n1, 1e-8) * jnp.maximum(n2, 1e-8))
    out_ref[...] = w * x2 + (1.0 - w) * x1


def _tc_final(p2, degp, x1):
    return pl.pallas_call(
        _final_body,
        grid=(NBLK,),
        in_specs=[
            pl.BlockSpec((NC, RB, D), lambda i: (0, i, 0)),
            pl.BlockSpec((NC, NPAD), lambda i: (0, 0)),
            pl.BlockSpec((RB, D), lambda i: (i, 0)),
        ],
        out_specs=pl.BlockSpec((RB, D), lambda i: (i, 0)),
        out_shape=jax.ShapeDtypeStruct((NPAD, D), jnp.float32),
    )(p2, degp, x1)


def kernel(features, edge_index):
    src3 = edge_index[0].reshape(NW, CW, K)
    dst3 = edge_index[1].reshape(NW, CW, K)
    feat_pad = jnp.pad(features, ((0, NPAD - N), (0, 0)))
    zrows = jnp.zeros((RT, D), jnp.float32)
    zdeg = jnp.zeros((RT,), jnp.float32)
    ones = jnp.ones((K,), jnp.float32)

    p1, degp = _sc_agg(feat_pad, src3, dst3, zrows, zdeg, ones)
    x1 = _tc_combine(p1, degp)
    p2, _ = _sc_agg(x1, src3, dst3, zrows, zdeg, ones)
    out = _tc_final(p2, degp, x1)
    return out[:N]


# speedup vs baseline: 12.3008x; 1.6387x over previous
import functools

import jax
import jax.numpy as jnp
from jax import lax
from jax.experimental import pallas as pl
from jax.experimental.pallas import tpu as pltpu
from jax.experimental.pallas import tpu_sc as plsc

N = 10000
E = 320000
D = 128

NC = 2
NS = 16
NW = NC * NS

NPAD = 10240
RT = NPAD // NS
K = 80
EW = E // NW
CW = EW // K

RB = 1024
NBLK = NPAD // RB


NBUF = 2


def _agg_body(x_hbm, src_hbm, dst_hbm, zrows_hbm, zdeg_hbm, ones_hbm,
              p_out, degp_out,
              acc_sh, deg_sh, srcbuf, dstbuf, rows_v, ones_v,
              gsem, ssem, dsem):
    cid = lax.axis_index("c")
    sid = lax.axis_index("s")
    wid = sid * NC + cid

    pltpu.sync_copy(zrows_hbm, acc_sh.at[pl.ds(sid * RT, RT)])
    pltpu.sync_copy(zdeg_hbm, deg_sh.at[pl.ds(sid * RT, RT)])
    pltpu.sync_copy(ones_hbm, ones_v)
    pltpu.sync_copy(src_hbm.at[wid], srcbuf)
    pltpu.sync_copy(dst_hbm.at[wid], dstbuf)
    plsc.subcore_barrier()

    for b in range(NBUF - 1):
        pltpu.async_copy(x_hbm.at[srcbuf.at[pl.ds(b * K, K)]], rows_v.at[b], gsem)

    def chunk(ci, carry):
        b = lax.rem(ci, NBUF)
        pb = lax.rem(ci + NBUF - 1, NBUF)

        @pl.when(ci > 0)
        def _():
            pltpu.make_async_copy(
                rows_v.at[pb], acc_sh.at[dstbuf.at[0]], ssem).wait()
            pltpu.make_async_copy(
                ones_v, deg_sh.at[dstbuf.at[0]], dsem).wait()

        @pl.when(ci + NBUF - 1 < CW)
        def _():
            pltpu.async_copy(
                x_hbm.at[srcbuf.at[pl.ds((ci + NBUF - 1) * K, K)]], rows_v.at[pb], gsem)

        pltpu.make_async_copy(
            x_hbm.at[srcbuf.at[pl.ds(ci * K, K)]], rows_v.at[b], gsem).wait()
        pltpu.async_copy(rows_v.at[b], acc_sh.at[dstbuf.at[ci]], ssem,
                         add=True)
        pltpu.async_copy(ones_v, deg_sh.at[dstbuf.at[ci]], dsem, add=True)
        return carry

    lax.fori_loop(0, CW, chunk, 0)
    pltpu.make_async_copy(rows_v.at[0], acc_sh.at[dstbuf.at[0]], ssem).wait()
    pltpu.make_async_copy(ones_v, deg_sh.at[dstbuf.at[0]], dsem).wait()
    plsc.subcore_barrier()

    pltpu.sync_copy(acc_sh.at[pl.ds(sid * RT, RT)],
                    p_out.at[cid, pl.ds(sid * RT, RT)])
    pltpu.sync_copy(deg_sh.at[pl.ds(sid * RT, RT)],
                    degp_out.at[cid, pl.ds(sid * RT, RT)])


_sc_agg = functools.partial(
    pl.kernel,
    out_type=(jax.ShapeDtypeStruct((NC, NPAD, D), jnp.float32),
              jax.ShapeDtypeStruct((NC, NPAD), jnp.float32)),
    mesh=plsc.VectorSubcoreMesh(core_axis_name="c", subcore_axis_name="s"),
    scratch_types=(
        pltpu.VMEM_SHARED((NPAD, D), jnp.float32),
        pltpu.VMEM_SHARED((NPAD,), jnp.float32),
        pltpu.VMEM((EW,), jnp.int32),
        pltpu.VMEM((CW, K), jnp.int32),
        pltpu.VMEM((NBUF, K, D), jnp.float32),
        pltpu.VMEM((K,), jnp.float32),
        pltpu.SemaphoreType.DMA,
        pltpu.SemaphoreType.DMA,
        pltpu.SemaphoreType.DMA,
    ),
)(_agg_body)


def _combine_body(p_ref, degp_ref, x1_ref):
    i = pl.program_id(0)
    deg = degp_ref[0, pl.ds(i * RB, RB)] + degp_ref[1, pl.ds(i * RB, RB)]
    rec = 1.0 / jnp.maximum(deg, 1.0)
    x1_ref[...] = (p_ref[0] + p_ref[1]) * rec[:, None]


def _tc_combine(p, degp):
    return pl.pallas_call(
        _combine_body,
        grid=(NBLK,),
        in_specs=[
            pl.BlockSpec((NC, RB, D), lambda i: (0, i, 0)),
            pl.BlockSpec((NC, NPAD), lambda i: (0, 0)),
        ],
        out_specs=pl.BlockSpec((RB, D), lambda i: (i, 0)),
        out_shape=jax.ShapeDtypeStruct((NPAD, D), jnp.float32),
    )(p, degp)


def _final_body(p_ref, degp_ref, x1_ref, out_ref):
    i = pl.program_id(0)
    deg = degp_ref[0, pl.ds(i * RB, RB)] + degp_ref[1, pl.ds(i * RB, RB)]
    rec = 1.0 / jnp.maximum(deg, 1.0)
    x2 = (p_ref[0] + p_ref[1]) * rec[:, None]
    x1 = x1_ref[...]
    dot = jnp.sum(x1 * x2, axis=1, keepdims=True)
    n1 = jnp.sqrt(jnp.sum(x1 * x1, axis=1, keepdims=True))
    n2 = jnp.sqrt(jnp.sum(x2 * x2, axis=1, keepdims=True))
    w = dot / (jnp.maximum(n1, 1e-8) * jnp.maximum(n2, 1e-8))
    out_ref[...] = w * x2 + (1.0 - w) * x1


def _tc_final(p2, degp, x1):
    return pl.pallas_call(
        _final_body,
        grid=(NBLK,),
        in_specs=[
            pl.BlockSpec((NC, RB, D), lambda i: (0, i, 0)),
            pl.BlockSpec((NC, NPAD), lambda i: (0, 0)),
            pl.BlockSpec((RB, D), lambda i: (i, 0)),
        ],
        out_specs=pl.BlockSpec((RB, D), lambda i: (i, 0)),
        out_shape=jax.ShapeDtypeStruct((NPAD, D), jnp.float32),
    )(p2, degp, x1)


def kernel(features, edge_index):
    src2 = edge_index[0].reshape(NW, EW)
    dst3 = edge_index[1].reshape(NW, CW, K)
    feat_pad = jnp.pad(features, ((0, NPAD - N), (0, 0)))
    zrows = jnp.zeros((RT, D), jnp.float32)
    zdeg = jnp.zeros((RT,), jnp.float32)
    ones = jnp.ones((K,), jnp.float32)

    p1, degp = _sc_agg(feat_pad, src2, dst3, zrows, zdeg, ones)
    x1 = _tc_combine(p1, degp)
    p2, _ = _sc_agg(x1, src2, dst3, zrows, zdeg, ones)
    out = _tc_final(p2, degp, x1)
    return out[:N]


# speedup vs baseline: 12.7767x; 1.0387x over previous
import functools

import jax
import jax.numpy as jnp
from jax import lax
from jax.experimental import pallas as pl
from jax.experimental.pallas import tpu as pltpu
from jax.experimental.pallas import tpu_sc as plsc

N = 10000
E = 320000
D = 128

NC = 2
NS = 16
NW = NC * NS

NPAD = 10240
RT = NPAD // NS
K = 80
EW = E // NW
CW = EW // K
NBUF = 2

RB = 1024
NBLK = NPAD // RB
RBO = RB
NBLKO = NBLK


def _make_agg(with_deg, n_rows):
    def body(*args):
        if with_deg:
            (x_hbm, src_hbm, dst_hbm, zrows_hbm, zdeg_hbm, ones_hbm,
             p_out, degp_out,
             acc_sh, deg_sh, srcbuf, dstbuf, rows_v, ones_v,
             gsem, ssem, dsem) = args
        else:
            (x_hbm, src_hbm, dst_hbm, zrows_hbm,
             p_out,
             acc_sh, srcbuf, dstbuf, rows_v,
             gsem, ssem) = args
        cid = lax.axis_index("c")
        sid = lax.axis_index("s")
        wid = sid * NC + cid

        pltpu.sync_copy(zrows_hbm, acc_sh.at[pl.ds(sid * RT, RT)])
        if with_deg:
            pltpu.sync_copy(zdeg_hbm, deg_sh.at[pl.ds(sid * RT, RT)])
            pltpu.sync_copy(ones_hbm, ones_v)
        pltpu.sync_copy(src_hbm.at[wid], srcbuf)
        pltpu.sync_copy(dst_hbm.at[wid], dstbuf)
        plsc.subcore_barrier()

        for b in range(NBUF - 1):
            pltpu.async_copy(
                x_hbm.at[srcbuf.at[pl.ds(b * K, K)]], rows_v.at[b], gsem)

        def chunk(ci, carry):
            b = lax.rem(ci, NBUF)
            pb = lax.rem(ci + NBUF - 1, NBUF)

            @pl.when(ci > 0)
            def _():
                pltpu.make_async_copy(
                    rows_v.at[pb], acc_sh.at[dstbuf.at[0]], ssem).wait()
                if with_deg:
                    pltpu.make_async_copy(
                        ones_v, deg_sh.at[dstbuf.at[0]], dsem).wait()

            @pl.when(ci + NBUF - 1 < CW)
            def _():
                pltpu.async_copy(
                    x_hbm.at[srcbuf.at[pl.ds((ci + NBUF - 1) * K, K)]],
                    rows_v.at[pb], gsem)

            pltpu.make_async_copy(
                x_hbm.at[srcbuf.at[pl.ds(ci * K, K)]], rows_v.at[b],
                gsem).wait()
            pltpu.async_copy(rows_v.at[b], acc_sh.at[dstbuf.at[ci]], ssem,
                             add=True)
            if with_deg:
                pltpu.async_copy(ones_v, deg_sh.at[dstbuf.at[ci]], dsem,
                                 add=True)
            return carry

        lax.fori_loop(0, CW, chunk, 0)
        pltpu.make_async_copy(
            rows_v.at[0], acc_sh.at[dstbuf.at[0]], ssem).wait()
        if with_deg:
            pltpu.make_async_copy(
                ones_v, deg_sh.at[dstbuf.at[0]], dsem).wait()
        plsc.subcore_barrier()

        pltpu.sync_copy(acc_sh.at[pl.ds(sid * RT, RT)],
                        p_out.at[cid, pl.ds(sid * RT, RT)])
        if with_deg:
            pltpu.sync_copy(deg_sh.at[pl.ds(sid * RT, RT)],
                            degp_out.at[cid, pl.ds(sid * RT, RT)])

    if with_deg:
        out_type = (jax.ShapeDtypeStruct((NC, NPAD, D), jnp.float32),
                    jax.ShapeDtypeStruct((NC, NPAD), jnp.float32))
    else:
        out_type = jax.ShapeDtypeStruct((NC, NPAD, D), jnp.float32)
    scratch = [pltpu.VMEM_SHARED((NPAD, D), jnp.float32)]
    if with_deg:
        scratch.append(pltpu.VMEM_SHARED((NPAD,), jnp.float32))
    scratch += [
        pltpu.VMEM((EW,), jnp.int32),
        pltpu.VMEM((CW, K), jnp.int32),
        pltpu.VMEM((NBUF, K, D), jnp.float32),
    ]
    if with_deg:
        scratch.append(pltpu.VMEM((K,), jnp.float32))
    scratch += [pltpu.SemaphoreType.DMA, pltpu.SemaphoreType.DMA]
    if with_deg:
        scratch.append(pltpu.SemaphoreType.DMA)
    return pl.kernel(
        body,
        out_type=out_type,
        mesh=plsc.VectorSubcoreMesh(core_axis_name="c", subcore_axis_name="s"),
        scratch_types=tuple(scratch),
    )


_sc_agg1 = _make_agg(True, N)
_sc_agg2 = _make_agg(False, NPAD)


def _combine_body(p_ref, degp_ref, x1_ref):
    i = pl.program_id(0)
    deg = degp_ref[0, pl.ds(i * RB, RB)] + degp_ref[1, pl.ds(i * RB, RB)]
    rec = 1.0 / jnp.maximum(deg, 1.0)
    x1_ref[...] = (p_ref[0] + p_ref[1]) * rec[:, None]


def _tc_combine(p, degp):
    return pl.pallas_call(
        _combine_body,
        grid=(NBLK,),
        in_specs=[
            pl.BlockSpec((NC, RB, D), lambda i: (0, i, 0)),
            pl.BlockSpec((NC, NPAD), lambda i: (0, 0)),
        ],
        out_specs=pl.BlockSpec((RB, D), lambda i: (i, 0)),
        out_shape=jax.ShapeDtypeStruct((NPAD, D), jnp.float32),
    )(p, degp)


def _final_body(p_ref, degp_ref, x1_ref, out_ref):
    i = pl.program_id(0)
    deg = degp_ref[0, pl.ds(i * RBO, RBO)] + degp_ref[1, pl.ds(i * RBO, RBO)]
    rec = 1.0 / jnp.maximum(deg, 1.0)
    x2 = (p_ref[0] + p_ref[1]) * rec[:, None]
    x1 = x1_ref[...]
    dot = jnp.sum(x1 * x2, axis=1, keepdims=True)
    n1 = jnp.sqrt(jnp.sum(x1 * x1, axis=1, keepdims=True))
    n2 = jnp.sqrt(jnp.sum(x2 * x2, axis=1, keepdims=True))
    w = dot / (jnp.maximum(n1, 1e-8) * jnp.maximum(n2, 1e-8))
    out_ref[...] = w * x2 + (1.0 - w) * x1


def _tc_final(p2, degp, x1):
    return pl.pallas_call(
        _final_body,
        grid=(NBLKO,),
        in_specs=[
            pl.BlockSpec((NC, RBO, D), lambda i: (0, i, 0)),
            pl.BlockSpec((NC, NPAD), lambda i: (0, 0)),
            pl.BlockSpec((RBO, D), lambda i: (i, 0)),
        ],
        out_specs=pl.BlockSpec((RBO, D), lambda i: (i, 0)),
        out_shape=jax.ShapeDtypeStruct((N, D), jnp.float32),
    )(p2, degp, x1)


def kernel(features, edge_index):
    src2 = edge_index[0].reshape(NW, EW)
    dst3 = edge_index[1].reshape(NW, CW, K)
    zrows = jnp.zeros((RT, D), jnp.float32)
    zdeg = jnp.zeros((RT,), jnp.float32)
    ones = jnp.ones((K,), jnp.float32)

    p1, degp = _sc_agg1(features, src2, dst3, zrows, zdeg, ones)
    x1 = _tc_combine(p1, degp)
    p2 = _sc_agg2(x1, src2, dst3, zrows)
    return _tc_final(p2, degp, x1)


# speedup vs baseline: 14.3984x; 1.1269x over previous
import functools

import jax
import jax.numpy as jnp
from jax import lax
from jax.experimental import pallas as pl
from jax.experimental.pallas import tpu as pltpu
from jax.experimental.pallas import tpu_sc as plsc

N = 10000
E = 320000
D = 128

NC = 2
NS = 16
NW = NC * NS

NPAD = 10240
RT = NPAD // NS
K = 80
EW = E // NW
CW = EW // K
NBUF = 4
GLEAD = 2
SLAG = 2
DG = 5
DB = CW // DG

RB = 1024
NBLK = NPAD // RB
RBO = RB
NBLKO = NBLK


def _make_agg(with_deg):
    def body(*args):
        if with_deg:
            (x_hbm, idx_hbm, zrows_hbm, zdeg_hbm, ones_hbm,
             p_out, degp_out,
             acc_sh, deg_sh, ibuf, rows_v, ones_v,
             gsem, ssem, dsem, isem) = args
        else:
            (x_hbm, idx_hbm, zrows_hbm,
             p_out,
             acc_sh, ibuf, rows_v,
             gsem, ssem, isem) = args
        cid = lax.axis_index("c")
        sid = lax.axis_index("s")
        wid = sid * NC + cid

        pltpu.sync_copy(zrows_hbm, acc_sh.at[pl.ds(sid * RT, RT)])
        if with_deg:
            pltpu.sync_copy(zdeg_hbm, deg_sh.at[pl.ds(sid * RT, RT)])
            pltpu.sync_copy(ones_hbm, ones_v)
        pltpu.sync_copy(idx_hbm.at[wid, 0], ibuf.at[0])
        plsc.subcore_barrier()

        def gather_idx(cn):
            return ibuf.at[lax.rem(lax.div(cn, DG), 2), 0, lax.rem(cn, DG)]

        for cn in range(GLEAD):
            pltpu.async_copy(x_hbm.at[gather_idx(cn)], rows_v.at[cn], gsem)

        def chunk(ci, carry):
            b = lax.rem(ci, NBUF)
            g = lax.div(ci, DG)
            r = lax.rem(ci, DG)
            gb = lax.rem(g, 2)

            @pl.when(ci >= SLAG)
            def _():
                pltpu.make_async_copy(
                    rows_v.at[0], acc_sh.at[ibuf.at[0, 1, 0]], ssem).wait()
                if with_deg:
                    pltpu.make_async_copy(
                        ones_v, deg_sh.at[ibuf.at[0, 1, 0]], dsem).wait()

            @pl.when(jnp.logical_and(r == 2, g + 1 < DB))
            def _():
                pltpu.async_copy(idx_hbm.at[wid, g + 1],
                                 ibuf.at[1 - gb], isem)

            @pl.when(jnp.logical_and(r == 3, g + 1 < DB))
            def _():
                pltpu.make_async_copy(idx_hbm.at[wid, 0],
                                      ibuf.at[0], isem).wait()

            @pl.when(ci + GLEAD < CW)
            def _():
                pltpu.async_copy(x_hbm.at[gather_idx(ci + GLEAD)],
                                 rows_v.at[lax.rem(ci + GLEAD, NBUF)], gsem)

            pltpu.make_async_copy(
                x_hbm.at[gather_idx(ci)], rows_v.at[b], gsem).wait()
            didx = ibuf.at[gb, 1, r]
            pltpu.async_copy(rows_v.at[b], acc_sh.at[didx], ssem, add=True)
            if with_deg:
                pltpu.async_copy(ones_v, deg_sh.at[didx], dsem, add=True)
            return carry

        lax.fori_loop(0, CW, chunk, 0)
        for _ in range(SLAG):
            pltpu.make_async_copy(
                rows_v.at[0], acc_sh.at[ibuf.at[0, 1, 0]], ssem).wait()
            if with_deg:
                pltpu.make_async_copy(
                    ones_v, deg_sh.at[ibuf.at[0, 1, 0]], dsem).wait()
        plsc.subcore_barrier()

        pltpu.sync_copy(acc_sh.at[pl.ds(sid * RT, RT)],
                        p_out.at[cid, pl.ds(sid * RT, RT)])
        if with_deg:
            pltpu.sync_copy(deg_sh.at[pl.ds(sid * RT, RT)],
                            degp_out.at[cid, pl.ds(sid * RT, RT)])

    if with_deg:
        out_type = (jax.ShapeDtypeStruct((NC, NPAD, D), jnp.float32),
                    jax.ShapeDtypeStruct((NC, NPAD), jnp.float32))
    else:
        out_type = jax.ShapeDtypeStruct((NC, NPAD, D), jnp.float32)
    scratch = [pltpu.VMEM_SHARED((NPAD, D), jnp.float32)]
    if with_deg:
        scratch.append(pltpu.VMEM_SHARED((NPAD,), jnp.float32))
    scratch += [
        pltpu.VMEM((2, 2, DG, K), jnp.int32),
        pltpu.VMEM((NBUF, K, D), jnp.float32),
    ]
    if with_deg:
        scratch.append(pltpu.VMEM((K,), jnp.float32))
    scratch += [pltpu.SemaphoreType.DMA, pltpu.SemaphoreType.DMA]
    if with_deg:
        scratch.append(pltpu.SemaphoreType.DMA)
    scratch.append(pltpu.SemaphoreType.DMA)
    return pl.kernel(
        body,
        out_type=out_type,
        mesh=plsc.VectorSubcoreMesh(core_axis_name="c", subcore_axis_name="s"),
        scratch_types=tuple(scratch),
    )


_sc_agg1 = _make_agg(True)
_sc_agg2 = _make_agg(False)


def _combine_body(p_ref, degp_ref, x1_ref):
    i = pl.program_id(0)
    deg = degp_ref[0, pl.ds(i * RB, RB)] + degp_ref[1, pl.ds(i * RB, RB)]
    rec = 1.0 / jnp.maximum(deg, 1.0)
    x1_ref[...] = (p_ref[0] + p_ref[1]) * rec[:, None]


def _tc_combine(p, degp):
    return pl.pallas_call(
        _combine_body,
        grid=(NBLK,),
        in_specs=[
            pl.BlockSpec((NC, RB, D), lambda i: (0, i, 0)),
            pl.BlockSpec((NC, NPAD), lambda i: (0, 0)),
        ],
        out_specs=pl.BlockSpec((RB, D), lambda i: (i, 0)),
        out_shape=jax.ShapeDtypeStruct((NPAD, D), jnp.float32),
    )(p, degp)


def _final_body(p_ref, degp_ref, x1_ref, out_ref):
    i = pl.program_id(0)
    deg = degp_ref[0, pl.ds(i * RBO, RBO)] + degp_ref[1, pl.ds(i * RBO, RBO)]
    rec = 1.0 / jnp.maximum(deg, 1.0)
    x2 = (p_ref[0] + p_ref[1]) * rec[:, None]
    x1 = x1_ref[...]
    dot = jnp.sum(x1 * x2, axis=1, keepdims=True)
    n1 = jnp.sqrt(jnp.sum(x1 * x1, axis=1, keepdims=True))
    n2 = jnp.sqrt(jnp.sum(x2 * x2, axis=1, keepdims=True))
    w = dot / (jnp.maximum(n1, 1e-8) * jnp.maximum(n2, 1e-8))
    out_ref[...] = w * x2 + (1.0 - w) * x1


def _tc_final(p2, degp, x1):
    return pl.pallas_call(
        _final_body,
        grid=(NBLKO,),
        in_specs=[
            pl.BlockSpec((NC, RBO, D), lambda i: (0, i, 0)),
            pl.BlockSpec((NC, NPAD), lambda i: (0, 0)),
            pl.BlockSpec((RBO, D), lambda i: (i, 0)),
        ],
        out_specs=pl.BlockSpec((RBO, D), lambda i: (i, 0)),
        out_shape=jax.ShapeDtypeStruct((N, D), jnp.float32),
    )(p2, degp, x1)


def kernel(features, edge_index):
    src4 = edge_index[0].reshape(NW, DB, DG, K)
    dst4 = edge_index[1].reshape(NW, DB, DG, K)
    idx5 = jnp.stack([src4, dst4], axis=2)
    zrows = jnp.zeros((RT, D), jnp.float32)
    zdeg = jnp.zeros((RT,), jnp.float32)
    ones = jnp.ones((K,), jnp.float32)

    p1, degp = _sc_agg1(features, idx5, zrows, zdeg, ones)
    x1 = _tc_combine(p1, degp)
    p2 = _sc_agg2(x1, idx5, zrows)
    return _tc_final(p2, degp, x1)


# speedup vs baseline: 14.5884x; 1.0132x over previous
import functools

import jax
import jax.numpy as jnp
from jax import lax
from jax.experimental import pallas as pl
from jax.experimental.pallas import tpu as pltpu
from jax.experimental.pallas import tpu_sc as plsc

N = 10000
E = 320000
D = 128

NC = 2
NS = 16
NW = NC * NS

NPAD = 10240
RT = NPAD // NS
K = 80
EW = E // NW
CW = EW // K
NBUF = 4
GLEAD = 2
SLAG = 2
DG = 5
DB = CW // DG

RB = 1024
NBLK = NPAD // RB
RBO = RB
NBLKO = NBLK


def _make_agg(with_deg):
    def body(*args):
        if with_deg:
            (x_hbm, src_hbm, dst_hbm, zrows_hbm, zdeg_hbm, ones_hbm,
             p_out, degp_out,
             acc_sh, deg_sh, ibuf, rows_v, ones_v,
             gsem, ssem, dsem, isem) = args
        else:
            (x_hbm, src_hbm, dst_hbm, zrows_hbm,
             p_out,
             acc_sh, ibuf, rows_v,
             gsem, ssem, isem) = args
        cid = lax.axis_index("c")
        sid = lax.axis_index("s")
        wid = sid * NC + cid

        pltpu.sync_copy(zrows_hbm, acc_sh.at[pl.ds(sid * RT, RT)])
        if with_deg:
            pltpu.sync_copy(zdeg_hbm, deg_sh.at[pl.ds(sid * RT, RT)])
            pltpu.sync_copy(ones_hbm, ones_v)
        pltpu.sync_copy(src_hbm.at[wid, 0], ibuf.at[0, 0])
        pltpu.sync_copy(dst_hbm.at[wid, 0], ibuf.at[0, 1])
        plsc.subcore_barrier()

        def gather_idx(cn):
            return ibuf.at[lax.rem(lax.div(cn, DG), 2), 0, lax.rem(cn, DG)]

        for cn in range(GLEAD):
            pltpu.async_copy(x_hbm.at[gather_idx(cn)], rows_v.at[cn], gsem)

        def chunk(ci, carry):
            b = lax.rem(ci, NBUF)
            g = lax.div(ci, DG)
            r = lax.rem(ci, DG)
            gb = lax.rem(g, 2)

            @pl.when(ci >= SLAG)
            def _():
                pltpu.make_async_copy(
                    rows_v.at[0], acc_sh.at[ibuf.at[0, 1, 0]], ssem).wait()
                if with_deg:
                    pltpu.make_async_copy(
                        ones_v, deg_sh.at[ibuf.at[0, 1, 0]], dsem).wait()

            @pl.when(jnp.logical_and(r == 2, g + 1 < DB))
            def _():
                pltpu.async_copy(src_hbm.at[wid, g + 1],
                                 ibuf.at[1 - gb, 0], isem)
                pltpu.async_copy(dst_hbm.at[wid, g + 1],
                                 ibuf.at[1 - gb, 1], isem)

            @pl.when(jnp.logical_and(r == 3, g + 1 < DB))
            def _():
                pltpu.make_async_copy(src_hbm.at[wid, 0],
                                      ibuf.at[0, 0], isem).wait()
                pltpu.make_async_copy(dst_hbm.at[wid, 0],
                                      ibuf.at[0, 1], isem).wait()

            @pl.when(ci + GLEAD < CW)
            def _():
                pltpu.async_copy(x_hbm.at[gather_idx(ci + GLEAD)],
                                 rows_v.at[lax.rem(ci + GLEAD, NBUF)], gsem)

            pltpu.make_async_copy(
                x_hbm.at[gather_idx(ci)], rows_v.at[b], gsem).wait()
            didx = ibuf.at[gb, 1, r]
            pltpu.async_copy(rows_v.at[b], acc_sh.at[didx], ssem, add=True)
            if with_deg:
                pltpu.async_copy(ones_v, deg_sh.at[didx], dsem, add=True)
            return carry

        lax.fori_loop(0, CW, chunk, 0)
        for _ in range(SLAG):
            pltpu.make_async_copy(
                rows_v.at[0], acc_sh.at[ibuf.at[0, 1, 0]], ssem).wait()
            if with_deg:
                pltpu.make_async_copy(
                    ones_v, deg_sh.at[ibuf.at[0, 1, 0]], dsem).wait()
        plsc.subcore_barrier()

        pltpu.sync_copy(acc_sh.at[pl.ds(sid * RT, RT)],
                        p_out.at[cid, pl.ds(sid * RT, RT)])
        if with_deg:
            pltpu.sync_copy(deg_sh.at[pl.ds(sid * RT, RT)],
                            degp_out.at[cid, pl.ds(sid * RT, RT)])

    if with_deg:
        out_type = (jax.ShapeDtypeStruct((NC, NPAD, D), jnp.float32),
                    jax.ShapeDtypeStruct((NC, NPAD), jnp.float32))
    else:
        out_type = jax.ShapeDtypeStruct((NC, NPAD, D), jnp.float32)
    scratch = [pltpu.VMEM_SHARED((NPAD, D), jnp.float32)]
    if with_deg:
        scratch.append(pltpu.VMEM_SHARED((NPAD,), jnp.float32))
    scratch += [
        pltpu.VMEM((2, 2, DG, K), jnp.int32),
        pltpu.VMEM((NBUF, K, D), jnp.float32),
    ]
    if with_deg:
        scratch.append(pltpu.VMEM((K,), jnp.float32))
    scratch += [pltpu.SemaphoreType.DMA, pltpu.SemaphoreType.DMA]
    if with_deg:
        scratch.append(pltpu.SemaphoreType.DMA)
    scratch.append(pltpu.SemaphoreType.DMA)
    return pl.kernel(
        body,
        out_type=out_type,
        mesh=plsc.VectorSubcoreMesh(core_axis_name="c", subcore_axis_name="s"),
        scratch_types=tuple(scratch),
    )


_sc_agg1 = _make_agg(True)
_sc_agg2 = _make_agg(False)


def _combine_body(p_ref, degp_ref, x1_ref):
    i = pl.program_id(0)
    deg = degp_ref[0, pl.ds(i * RB, RB)] + degp_ref[1, pl.ds(i * RB, RB)]
    rec = 1.0 / jnp.maximum(deg, 1.0)
    x1_ref[...] = (p_ref[0] + p_ref[1]) * rec[:, None]


def _tc_combine(p, degp):
    return pl.pallas_call(
        _combine_body,
        grid=(NBLK,),
        in_specs=[
            pl.BlockSpec((NC, RB, D), lambda i: (0, i, 0)),
            pl.BlockSpec((NC, NPAD), lambda i: (0, 0)),
        ],
        out_specs=pl.BlockSpec((RB, D), lambda i: (i, 0)),
        out_shape=jax.ShapeDtypeStruct((NPAD, D), jnp.float32),
    )(p, degp)


def _final_body(p_ref, degp_ref, x1_ref, out_ref):
    i = pl.program_id(0)
    deg = degp_ref[0, pl.ds(i * RBO, RBO)] + degp_ref[1, pl.ds(i * RBO, RBO)]
    rec = 1.0 / jnp.maximum(deg, 1.0)
    x2 = (p_ref[0] + p_ref[1]) * rec[:, None]
    x1 = x1_ref[...]
    dot = jnp.sum(x1 * x2, axis=1, keepdims=True)
    n1 = jnp.sqrt(jnp.sum(x1 * x1, axis=1, keepdims=True))
    n2 = jnp.sqrt(jnp.sum(x2 * x2, axis=1, keepdims=True))
    w = dot / (jnp.maximum(n1, 1e-8) * jnp.maximum(n2, 1e-8))
    out_ref[...] = w * x2 + (1.0 - w) * x1


def _tc_final(p2, degp, x1):
    return pl.pallas_call(
        _final_body,
        grid=(NBLKO,),
        in_specs=[
            pl.BlockSpec((NC, RBO, D), lambda i: (0, i, 0)),
            pl.BlockSpec((NC, NPAD), lambda i: (0, 0)),
            pl.BlockSpec((RBO, D), lambda i: (i, 0)),
        ],
        out_specs=pl.BlockSpec((RBO, D), lambda i: (i, 0)),
        out_shape=jax.ShapeDtypeStruct((N, D), jnp.float32),
    )(p2, degp, x1)


def kernel(features, edge_index):
    src4 = edge_index[0].reshape(NW, DB, DG, K)
    dst4 = edge_index[1].reshape(NW, DB, DG, K)
    zrows = jnp.zeros((RT, D), jnp.float32)
    zdeg = jnp.zeros((RT,), jnp.float32)
    ones = jnp.ones((K,), jnp.float32)

    p1, degp = _sc_agg1(features, src4, dst4, zrows, zdeg, ones)
    x1 = _tc_combine(p1, degp)
    p2 = _sc_agg2(x1, src4, dst4, zrows)
    return _tc_final(p2, degp, x1)


# speedup vs baseline: 16.0395x; 1.0995x over previous
import functools

import jax
import jax.numpy as jnp
from jax import lax
from jax.experimental import pallas as pl
from jax.experimental.pallas import tpu as pltpu
from jax.experimental.pallas import tpu_sc as plsc

N = 10000
E = 320000
D = 128

NC = 2
NS = 16
NW = NC * NS

NPAD = 10240
RT = NPAD // NS
K = 80
EW = E // NW
CW = EW // K
NBUF = 4
GLEAD = 2
SLAG = 2
DG = 5
DB = CW // DG

RB = 1024
NBLK = NPAD // RB
RBO = RB
NBLKO = NBLK


def _make_agg(with_deg):
    def body(*args):
        if with_deg:
            (x_hbm, edge_hbm,
             p_out, degp_out,
             acc_sh, deg_sh, ibuf, rows_v, ones_v,
             gsem, ssem, dsem, isem, zsem) = args
        else:
            (x_hbm, edge_hbm,
             p_out,
             acc_sh, ibuf, rows_v,
             gsem, ssem, isem, zsem) = args
        cid = lax.axis_index("c")
        sid = lax.axis_index("s")
        wid = sid * NC + cid

        pltpu.sync_copy(edge_hbm.at[0, wid, 0], ibuf.at[0, 0])
        pltpu.sync_copy(edge_hbm.at[1, wid, 0], ibuf.at[0, 1])

        z16 = jnp.zeros((16,), jnp.float32)

        def zfill(i, carry):
            for j in range(D // 16):
                rows_v[NBUF - 1, i, pl.ds(j * 16, 16)] = z16
            return carry

        lax.fori_loop(0, K, zfill, 0)
        if with_deg:
            one16 = jnp.full((16,), 1.0, jnp.float32)
            for j in range(K // 16):
                ones_v[pl.ds(j * 16, 16)] = one16
        zrow = rows_v.at[NBUF - 1]
        for t in range(RT // K):
            pltpu.async_copy(
                zrow, acc_sh.at[pl.ds(sid * RT + t * K, K)], zsem)
        if with_deg:
            for t in range(RT // D):
                pltpu.async_copy(
                    rows_v.at[NBUF - 1, 0],
                    deg_sh.at[pl.ds(sid * RT + t * D, D)], zsem)
        for t in range(RT // K):
            pltpu.make_async_copy(
                zrow, acc_sh.at[pl.ds(sid * RT, K)], zsem).wait()
        if with_deg:
            for t in range(RT // D):
                pltpu.make_async_copy(
                    rows_v.at[NBUF - 1, 0],
                    deg_sh.at[pl.ds(sid * RT, D)], zsem).wait()
        plsc.subcore_barrier()

        def gather_idx(cn):
            return ibuf.at[lax.rem(lax.div(cn, DG), 2), 0, lax.rem(cn, DG)]

        for cn in range(GLEAD):
            pltpu.async_copy(x_hbm.at[gather_idx(cn)], rows_v.at[cn], gsem)

        def chunk(ci, carry):
            b = lax.rem(ci, NBUF)
            g = lax.div(ci, DG)
            r = lax.rem(ci, DG)
            gb = lax.rem(g, 2)

            @pl.when(ci >= SLAG)
            def _():
                pltpu.make_async_copy(
                    rows_v.at[0], acc_sh.at[ibuf.at[0, 1, 0]], ssem).wait()
                if with_deg:
                    pltpu.make_async_copy(
                        ones_v, deg_sh.at[ibuf.at[0, 1, 0]], dsem).wait()

            @pl.when(jnp.logical_and(r == 2, g + 1 < DB))
            def _():
                pltpu.async_copy(edge_hbm.at[0, wid, g + 1],
                                 ibuf.at[1 - gb, 0], isem)
                pltpu.async_copy(edge_hbm.at[1, wid, g + 1],
                                 ibuf.at[1 - gb, 1], isem)

            @pl.when(jnp.logical_and(r == 3, g + 1 < DB))
            def _():
                pltpu.make_async_copy(edge_hbm.at[0, wid, 0],
                                      ibuf.at[0, 0], isem).wait()
                pltpu.make_async_copy(edge_hbm.at[1, wid, 0],
                                      ibuf.at[0, 1], isem).wait()

            @pl.when(ci + GLEAD < CW)
            def _():
                pltpu.async_copy(x_hbm.at[gather_idx(ci + GLEAD)],
                                 rows_v.at[lax.rem(ci + GLEAD, NBUF)], gsem)

            pltpu.make_async_copy(
                x_hbm.at[gather_idx(ci)], rows_v.at[b], gsem).wait()
            didx = ibuf.at[gb, 1, r]
            pltpu.async_copy(rows_v.at[b], acc_sh.at[didx], ssem, add=True)
            if with_deg:
                pltpu.async_copy(ones_v, deg_sh.at[didx], dsem, add=True)
            return carry

        lax.fori_loop(0, CW, chunk, 0)
        for _ in range(SLAG):
            pltpu.make_async_copy(
                rows_v.at[0], acc_sh.at[ibuf.at[0, 1, 0]], ssem).wait()
            if with_deg:
                pltpu.make_async_copy(
                    ones_v, deg_sh.at[ibuf.at[0, 1, 0]], dsem).wait()
        plsc.subcore_barrier()

        pltpu.sync_copy(acc_sh.at[pl.ds(sid * RT, RT)],
                        p_out.at[cid, pl.ds(sid * RT, RT)])
        if with_deg:
            pltpu.sync_copy(deg_sh.at[pl.ds(sid * RT, RT)],
                            degp_out.at[cid, pl.ds(sid * RT, RT)])

    if with_deg:
        out_type = (jax.ShapeDtypeStruct((NC, NPAD, D), jnp.float32),
                    jax.ShapeDtypeStruct((NC, NPAD), jnp.float32))
    else:
        out_type = jax.ShapeDtypeStruct((NC, NPAD, D), jnp.float32)
    scratch = [pltpu.VMEM_SHARED((NPAD, D), jnp.float32)]
    if with_deg:
        scratch.append(pltpu.VMEM_SHARED((NPAD,), jnp.float32))
    scratch += [
        pltpu.VMEM((2, 2, DG, K), jnp.int32),
        pltpu.VMEM((NBUF, K, D), jnp.float32),
    ]
    if with_deg:
        scratch.append(pltpu.VMEM((K,), jnp.float32))
    scratch += [pltpu.SemaphoreType.DMA, pltpu.SemaphoreType.DMA]
    if with_deg:
        scratch.append(pltpu.SemaphoreType.DMA)
    scratch.append(pltpu.SemaphoreType.DMA)
    scratch.append(pltpu.SemaphoreType.DMA)
    return pl.kernel(
        body,
        out_type=out_type,
        mesh=plsc.VectorSubcoreMesh(core_axis_name="c", subcore_axis_name="s"),
        scratch_types=tuple(scratch),
    )


_sc_agg1 = _make_agg(True)
_sc_agg2 = _make_agg(False)


def _combine_body(p_ref, degp_ref, x1_ref):
    i = pl.program_id(0)
    deg = degp_ref[0, pl.ds(i * RB, RB)] + degp_ref[1, pl.ds(i * RB, RB)]
    rec = 1.0 / jnp.maximum(deg, 1.0)
    x1_ref[...] = (p_ref[0] + p_ref[1]) * rec[:, None]


def _tc_combine(p, degp):
    return pl.pallas_call(
        _combine_body,
        grid=(NBLK,),
        in_specs=[
            pl.BlockSpec((NC, RB, D), lambda i: (0, i, 0)),
            pl.BlockSpec((NC, NPAD), lambda i: (0, 0)),
        ],
        out_specs=pl.BlockSpec((RB, D), lambda i: (i, 0)),
        out_shape=jax.ShapeDtypeStruct((NPAD, D), jnp.float32),
    )(p, degp)


def _final_body(p_ref, degp_ref, x1_ref, out_ref):
    i = pl.program_id(0)
    deg = degp_ref[0, pl.ds(i * RBO, RBO)] + degp_ref[1, pl.ds(i * RBO, RBO)]
    rec = 1.0 / jnp.maximum(deg, 1.0)
    x2 = (p_ref[0] + p_ref[1]) * rec[:, None]
    x1 = x1_ref[...]
    dot = jnp.sum(x1 * x2, axis=1, keepdims=True)
    n1 = jnp.sqrt(jnp.sum(x1 * x1, axis=1, keepdims=True))
    n2 = jnp.sqrt(jnp.sum(x2 * x2, axis=1, keepdims=True))
    w = dot / (jnp.maximum(n1, 1e-8) * jnp.maximum(n2, 1e-8))
    out_ref[...] = w * x2 + (1.0 - w) * x1


def _tc_final(p2, degp, x1):
    return pl.pallas_call(
        _final_body,
        grid=(NBLKO,),
        in_specs=[
            pl.BlockSpec((NC, RBO, D), lambda i: (0, i, 0)),
            pl.BlockSpec((NC, NPAD), lambda i: (0, 0)),
            pl.BlockSpec((RBO, D), lambda i: (i, 0)),
        ],
        out_specs=pl.BlockSpec((RBO, D), lambda i: (i, 0)),
        out_shape=jax.ShapeDtypeStruct((N, D), jnp.float32),
    )(p2, degp, x1)


def kernel(features, edge_index):
    edge5 = edge_index.reshape(2, NW, DB, DG, K)

    p1, degp = _sc_agg1(features, edge5)
    x1 = _tc_combine(p1, degp)
    p2 = _sc_agg2(x1, edge5)
    return _tc_final(p2, degp, x1)


# speedup vs baseline: 16.3256x; 1.0178x over previous
import functools

import jax
import jax.numpy as jnp
from jax import lax
from jax.experimental import pallas as pl
from jax.experimental.pallas import tpu as pltpu
from jax.experimental.pallas import tpu_sc as plsc

N = 10000
E = 320000
D = 128

NC = 2
NS = 16
NW = NC * NS

NPAD = 10240
RT = NPAD // NS
K = 80
EW = E // NW
CW = EW // K
NBUF = 4
GLEAD = 2
SLAG = 2
DG = 5
DB = CW // DG

RB = 2048
NBLK = NPAD // RB
RBO = RB
NBLKO = NBLK


def _make_agg(with_deg):
    def body(*args):
        if with_deg:
            (x_hbm, edge_hbm,
             p_out, degp_out,
             acc_sh, deg_sh, ibuf, rows_v, ones_v,
             gsem, ssem, dsem, isem, zsem) = args
        else:
            (x_hbm, edge_hbm,
             p_out,
             acc_sh, ibuf, rows_v,
             gsem, ssem, isem, zsem) = args
        cid = lax.axis_index("c")
        sid = lax.axis_index("s")
        wid = sid * NC + cid

        pltpu.sync_copy(edge_hbm.at[0, wid, 0], ibuf.at[0, 0])
        pltpu.sync_copy(edge_hbm.at[1, wid, 0], ibuf.at[0, 1])

        z16 = jnp.zeros((16,), jnp.float32)

        def zfill(i, carry):
            for j in range(D // 16):
                rows_v[NBUF - 1, i, pl.ds(j * 16, 16)] = z16
            return carry

        lax.fori_loop(0, K, zfill, 0)
        if with_deg:
            one16 = jnp.full((16,), 1.0, jnp.float32)
            for j in range(K // 16):
                ones_v[pl.ds(j * 16, 16)] = one16
        zrow = rows_v.at[NBUF - 1]
        for t in range(RT // K):
            pltpu.async_copy(
                zrow, acc_sh.at[pl.ds(sid * RT + t * K, K)], zsem)
        if with_deg:
            for t in range(RT // D):
                pltpu.async_copy(
                    rows_v.at[NBUF - 1, 0],
                    deg_sh.at[pl.ds(sid * RT + t * D, D)], zsem)
        for t in range(RT // K):
            pltpu.make_async_copy(
                zrow, acc_sh.at[pl.ds(sid * RT, K)], zsem).wait()
        if with_deg:
            for t in range(RT // D):
                pltpu.make_async_copy(
                    rows_v.at[NBUF - 1, 0],
                    deg_sh.at[pl.ds(sid * RT, D)], zsem).wait()
        plsc.subcore_barrier()

        def gather_idx(cn):
            return ibuf.at[lax.rem(lax.div(cn, DG), 2), 0, lax.rem(cn, DG)]

        for cn in range(GLEAD):
            pltpu.async_copy(x_hbm.at[gather_idx(cn)], rows_v.at[cn], gsem)

        def chunk(ci, carry):
            b = lax.rem(ci, NBUF)
            g = lax.div(ci, DG)
            r = lax.rem(ci, DG)
            gb = lax.rem(g, 2)

            @pl.when(ci >= SLAG)
            def _():
                pltpu.make_async_copy(
                    rows_v.at[0], acc_sh.at[ibuf.at[0, 1, 0]], ssem).wait()
                if with_deg:
                    pltpu.make_async_copy(
                        ones_v, deg_sh.at[ibuf.at[0, 1, 0]], dsem).wait()

            @pl.when(jnp.logical_and(r == 2, g + 1 < DB))
            def _():
                pltpu.async_copy(edge_hbm.at[0, wid, g + 1],
                                 ibuf.at[1 - gb, 0], isem)
                pltpu.async_copy(edge_hbm.at[1, wid, g + 1],
                                 ibuf.at[1 - gb, 1], isem)

            @pl.when(jnp.logical_and(r == 3, g + 1 < DB))
            def _():
                pltpu.make_async_copy(edge_hbm.at[0, wid, 0],
                                      ibuf.at[0, 0], isem).wait()
                pltpu.make_async_copy(edge_hbm.at[1, wid, 0],
                                      ibuf.at[0, 1], isem).wait()

            @pl.when(ci + GLEAD < CW)
            def _():
                pltpu.async_copy(x_hbm.at[gather_idx(ci + GLEAD)],
                                 rows_v.at[lax.rem(ci + GLEAD, NBUF)], gsem)

            pltpu.make_async_copy(
                x_hbm.at[gather_idx(ci)], rows_v.at[b], gsem).wait()
            didx = ibuf.at[gb, 1, r]
            pltpu.async_copy(rows_v.at[b], acc_sh.at[didx], ssem, add=True)
            if with_deg:
                pltpu.async_copy(ones_v, deg_sh.at[didx], dsem, add=True)
            return carry

        lax.fori_loop(0, CW, chunk, 0)
        for _ in range(SLAG):
            pltpu.make_async_copy(
                rows_v.at[0], acc_sh.at[ibuf.at[0, 1, 0]], ssem).wait()
            if with_deg:
                pltpu.make_async_copy(
                    ones_v, deg_sh.at[ibuf.at[0, 1, 0]], dsem).wait()
        plsc.subcore_barrier()

        pltpu.sync_copy(acc_sh.at[pl.ds(sid * RT, RT)],
                        p_out.at[cid, pl.ds(sid * RT, RT)])
        if with_deg:
            pltpu.sync_copy(deg_sh.at[pl.ds(sid * RT, RT)],
                            degp_out.at[cid, pl.ds(sid * RT, RT)])

    if with_deg:
        out_type = (jax.ShapeDtypeStruct((NC, NPAD, D), jnp.float32),
                    jax.ShapeDtypeStruct((NC, NPAD), jnp.float32))
    else:
        out_type = jax.ShapeDtypeStruct((NC, NPAD, D), jnp.float32)
    scratch = [pltpu.VMEM_SHARED((NPAD, D), jnp.float32)]
    if with_deg:
        scratch.append(pltpu.VMEM_SHARED((NPAD,), jnp.float32))
    scratch += [
        pltpu.VMEM((2, 2, DG, K), jnp.int32),
        pltpu.VMEM((NBUF, K, D), jnp.float32),
    ]
    if with_deg:
        scratch.append(pltpu.VMEM((K,), jnp.float32))
    scratch += [pltpu.SemaphoreType.DMA, pltpu.SemaphoreType.DMA]
    if with_deg:
        scratch.append(pltpu.SemaphoreType.DMA)
    scratch.append(pltpu.SemaphoreType.DMA)
    scratch.append(pltpu.SemaphoreType.DMA)
    return pl.kernel(
        body,
        out_type=out_type,
        mesh=plsc.VectorSubcoreMesh(core_axis_name="c", subcore_axis_name="s"),
        scratch_types=tuple(scratch),
    )


_sc_agg1 = _make_agg(True)
_sc_agg2 = _make_agg(False)


def _combine_body(p_ref, degp_ref, x1_ref):
    i = pl.program_id(0)
    deg = degp_ref[0, pl.ds(i * RB, RB)] + degp_ref[1, pl.ds(i * RB, RB)]
    rec = 1.0 / jnp.maximum(deg, 1.0)
    x1_ref[...] = (p_ref[0] + p_ref[1]) * rec[:, None]


def _tc_combine(p, degp):
    return pl.pallas_call(
        _combine_body,
        grid=(NBLK,),
        in_specs=[
            pl.BlockSpec((NC, RB, D), lambda i: (0, i, 0)),
            pl.BlockSpec((NC, NPAD), lambda i: (0, 0)),
        ],
        out_specs=pl.BlockSpec((RB, D), lambda i: (i, 0)),
        out_shape=jax.ShapeDtypeStruct((NPAD, D), jnp.float32),
    )(p, degp)


def _final_body(p_ref, degp_ref, x1_ref, out_ref):
    i = pl.program_id(0)
    deg = degp_ref[0, pl.ds(i * RBO, RBO)] + degp_ref[1, pl.ds(i * RBO, RBO)]
    rec = 1.0 / jnp.maximum(deg, 1.0)
    x2 = (p_ref[0] + p_ref[1]) * rec[:, None]
    x1 = x1_ref[...]
    dot = jnp.sum(x1 * x2, axis=1, keepdims=True)
    n1 = jnp.sqrt(jnp.sum(x1 * x1, axis=1, keepdims=True))
    n2 = jnp.sqrt(jnp.sum(x2 * x2, axis=1, keepdims=True))
    w = dot / (jnp.maximum(n1, 1e-8) * jnp.maximum(n2, 1e-8))
    out_ref[...] = w * x2 + (1.0 - w) * x1


def _tc_final(p2, degp, x1):
    return pl.pallas_call(
        _final_body,
        grid=(NBLKO,),
        in_specs=[
            pl.BlockSpec((NC, RBO, D), lambda i: (0, i, 0)),
            pl.BlockSpec((NC, NPAD), lambda i: (0, 0)),
            pl.BlockSpec((RBO, D), lambda i: (i, 0)),
        ],
        out_specs=pl.BlockSpec((RBO, D), lambda i: (i, 0)),
        out_shape=jax.ShapeDtypeStruct((N, D), jnp.float32),
    )(p2, degp, x1)


def kernel(features, edge_index):
    edge5 = edge_index.reshape(2, NW, DB, DG, K)

    p1, degp = _sc_agg1(features, edge5)
    x1 = _tc_combine(p1, degp)
    p2 = _sc_agg2(x1, edge5)
    return _tc_final(p2, degp, x1)


# speedup vs baseline: 16.4614x; 1.0083x over previous
import functools

import jax
import jax.numpy as jnp
from jax import lax
from jax.experimental import pallas as pl
from jax.experimental.pallas import tpu as pltpu
from jax.experimental.pallas import tpu_sc as plsc

N = 10000
E = 320000
D = 128

NC = 2
NS = 16
NW = NC * NS

NPAD = 10240
RT = NPAD // NS
K = 80
EW = E // NW
CW = EW // K
NBUF = 4
GLEAD = 2
SLAG = 2
DG = 5
DB = CW // DG

RB = 2048
NBLK = NPAD // RB
RBO = RB
NBLKO = NBLK


def _make_agg(with_deg):
    def body(*args):
        if with_deg:
            (x_hbm, edge_hbm,
             p_out, degp_out,
             acc_sh, deg_sh, ibuf, rows_v, ones_v,
             gsem, ssem, dsem, isem, zsem) = args
        else:
            (x_hbm, edge_hbm,
             p_out,
             acc_sh, ibuf, rows_v,
             gsem, ssem, isem, zsem) = args
        cid = lax.axis_index("c")
        sid = lax.axis_index("s")
        wid = sid * NC + cid

        pltpu.sync_copy(edge_hbm.at[0, wid, 0], ibuf.at[0, 0])
        pltpu.sync_copy(edge_hbm.at[1, wid, 0], ibuf.at[0, 1])

        def gather_idx(cn):
            return ibuf.at[lax.rem(lax.div(cn, DG), 2), 0, lax.rem(cn, DG)]

        for cn in range(GLEAD):
            pltpu.async_copy(x_hbm.at[gather_idx(cn)], rows_v.at[cn], gsem)

        z16 = jnp.zeros((16,), jnp.float32)

        def zfill(i, carry):
            for j in range(D // 16):
                rows_v[NBUF - 1, i, pl.ds(j * 16, 16)] = z16
            return carry

        lax.fori_loop(0, K, zfill, 0)
        if with_deg:
            one16 = jnp.full((16,), 1.0, jnp.float32)
            for j in range(K // 16):
                ones_v[pl.ds(j * 16, 16)] = one16
        zrow = rows_v.at[NBUF - 1]
        for t in range(RT // K):
            pltpu.async_copy(
                zrow, acc_sh.at[pl.ds(sid * RT + t * K, K)], zsem)
        if with_deg:
            for t in range(RT // D):
                pltpu.async_copy(
                    rows_v.at[NBUF - 1, 0],
                    deg_sh.at[pl.ds(sid * RT + t * D, D)], zsem)
        for t in range(RT // K):
            pltpu.make_async_copy(
                zrow, acc_sh.at[pl.ds(sid * RT, K)], zsem).wait()
        if with_deg:
            for t in range(RT // D):
                pltpu.make_async_copy(
                    rows_v.at[NBUF - 1, 0],
                    deg_sh.at[pl.ds(sid * RT, D)], zsem).wait()
        plsc.subcore_barrier()

        def chunk(ci, carry):
            b = lax.rem(ci, NBUF)
            g = lax.div(ci, DG)
            r = lax.rem(ci, DG)
            gb = lax.rem(g, 2)

            @pl.when(ci >= SLAG)
            def _():
                pltpu.make_async_copy(
                    rows_v.at[0], acc_sh.at[ibuf.at[0, 1, 0]], ssem).wait()
                if with_deg:
                    pltpu.make_async_copy(
                        ones_v, deg_sh.at[ibuf.at[0, 1, 0]], dsem).wait()

            @pl.when(jnp.logical_and(r == 2, g + 1 < DB))
            def _():
                pltpu.async_copy(edge_hbm.at[0, wid, g + 1],
                                 ibuf.at[1 - gb, 0], isem)
                pltpu.async_copy(edge_hbm.at[1, wid, g + 1],
                                 ibuf.at[1 - gb, 1], isem)

            @pl.when(jnp.logical_and(r == 3, g + 1 < DB))
            def _():
                pltpu.make_async_copy(edge_hbm.at[0, wid, 0],
                                      ibuf.at[0, 0], isem).wait()
                pltpu.make_async_copy(edge_hbm.at[1, wid, 0],
                                      ibuf.at[0, 1], isem).wait()

            @pl.when(ci + GLEAD < CW)
            def _():
                pltpu.async_copy(x_hbm.at[gather_idx(ci + GLEAD)],
                                 rows_v.at[lax.rem(ci + GLEAD, NBUF)], gsem)

            pltpu.make_async_copy(
                x_hbm.at[gather_idx(ci)], rows_v.at[b], gsem).wait()
            didx = ibuf.at[gb, 1, r]
            pltpu.async_copy(rows_v.at[b], acc_sh.at[didx], ssem, add=True)
            if with_deg:
                pltpu.async_copy(ones_v, deg_sh.at[didx], dsem, add=True)
            return carry

        lax.fori_loop(0, CW, chunk, 0)
        for _ in range(SLAG):
            pltpu.make_async_copy(
                rows_v.at[0], acc_sh.at[ibuf.at[0, 1, 0]], ssem).wait()
            if with_deg:
                pltpu.make_async_copy(
                    ones_v, deg_sh.at[ibuf.at[0, 1, 0]], dsem).wait()
        plsc.subcore_barrier()

        pltpu.sync_copy(acc_sh.at[pl.ds(sid * RT, RT)],
                        p_out.at[cid, pl.ds(sid * RT, RT)])
        if with_deg:
            pltpu.sync_copy(deg_sh.at[pl.ds(sid * RT, RT)],
                            degp_out.at[cid, pl.ds(sid * RT, RT)])

    if with_deg:
        out_type = (jax.ShapeDtypeStruct((NC, NPAD, D), jnp.float32),
                    jax.ShapeDtypeStruct((NC, NPAD), jnp.float32))
    else:
        out_type = jax.ShapeDtypeStruct((NC, NPAD, D), jnp.float32)
    scratch = [pltpu.VMEM_SHARED((NPAD, D), jnp.float32)]
    if with_deg:
        scratch.append(pltpu.VMEM_SHARED((NPAD,), jnp.float32))
    scratch += [
        pltpu.VMEM((2, 2, DG, K), jnp.int32),
        pltpu.VMEM((NBUF, K, D), jnp.float32),
    ]
    if with_deg:
        scratch.append(pltpu.VMEM((K,), jnp.float32))
    scratch += [pltpu.SemaphoreType.DMA, pltpu.SemaphoreType.DMA]
    if with_deg:
        scratch.append(pltpu.SemaphoreType.DMA)
    scratch.append(pltpu.SemaphoreType.DMA)
    scratch.append(pltpu.SemaphoreType.DMA)
    return pl.kernel(
        body,
        out_type=out_type,
        mesh=plsc.VectorSubcoreMesh(core_axis_name="c", subcore_axis_name="s"),
        scratch_types=tuple(scratch),
    )


_sc_agg1 = _make_agg(True)
_sc_agg2 = _make_agg(False)


def _combine_body(p_ref, degp_ref, x1_ref):
    i = pl.program_id(0)
    deg = degp_ref[0, pl.ds(i * RB, RB)] + degp_ref[1, pl.ds(i * RB, RB)]
    rec = 1.0 / jnp.maximum(deg, 1.0)
    x1_ref[...] = (p_ref[0] + p_ref[1]) * rec[:, None]


def _tc_combine(p, degp):
    return pl.pallas_call(
        _combine_body,
        grid=(NBLK,),
        in_specs=[
            pl.BlockSpec((NC, RB, D), lambda i: (0, i, 0)),
            pl.BlockSpec((NC, NPAD), lambda i: (0, 0)),
        ],
        out_specs=pl.BlockSpec((RB, D), lambda i: (i, 0)),
        out_shape=jax.ShapeDtypeStruct((NPAD, D), jnp.float32),
    )(p, degp)


def _final_body(p_ref, degp_ref, x1_ref, out_ref):
    i = pl.program_id(0)
    deg = degp_ref[0, pl.ds(i * RBO, RBO)] + degp_ref[1, pl.ds(i * RBO, RBO)]
    rec = 1.0 / jnp.maximum(deg, 1.0)
    x2 = (p_ref[0] + p_ref[1]) * rec[:, None]
    x1 = x1_ref[...]
    dot = jnp.sum(x1 * x2, axis=1, keepdims=True)
    n1 = jnp.sqrt(jnp.sum(x1 * x1, axis=1, keepdims=True))
    n2 = jnp.sqrt(jnp.sum(x2 * x2, axis=1, keepdims=True))
    w = dot / (jnp.maximum(n1, 1e-8) * jnp.maximum(n2, 1e-8))
    out_ref[...] = w * x2 + (1.0 - w) * x1


def _tc_final(p2, degp, x1):
    return pl.pallas_call(
        _final_body,
        grid=(NBLKO,),
        in_specs=[
            pl.BlockSpec((NC, RBO, D), lambda i: (0, i, 0)),
            pl.BlockSpec((NC, NPAD), lambda i: (0, 0)),
            pl.BlockSpec((RBO, D), lambda i: (i, 0)),
        ],
        out_specs=pl.BlockSpec((RBO, D), lambda i: (i, 0)),
        out_shape=jax.ShapeDtypeStruct((N, D), jnp.float32),
    )(p2, degp, x1)


def kernel(features, edge_index):
    edge5 = edge_index.reshape(2, NW, DB, DG, K)

    p1, degp = _sc_agg1(features, edge5)
    x1 = _tc_combine(p1, degp)
    p2 = _sc_agg2(x1, edge5)
    return _tc_final(p2, degp, x1)
